# baseline (speedup 1.0000x reference)
import functools

import jax
import jax.numpy as jnp
from jax import lax
from jax.experimental import pallas as pl
from jax.experimental.pallas import tpu as pltpu
from jax.experimental.pallas import tpu_sc as plsc

B = 1024
NUSERS = 16384
NNEWS = 100001
DIM = 64
TITLE = 768
NHOP = 2
NMEM = 32
KGE_W = 0.01
L2_W = 1e-07

NC = 2
NS = 16
NW = NC * NS
UPT = B // NW
NCAND = 5
CPT = NCAND * UPT

_ROLES = (
    ('ent', 1, 0),
    ('ent', 2, 0),
    ('rel', 0, 1),
    ('rel', 1, 1),
    ('rel', 2, 1),
    ('ent', 0, 2),
    ('ent', 1, 2),
    ('ent', 2, 2),
)


def _sc_gather_body(uidx_hbm, cand_hbm, ripple_hbm, news_hbm, ent_hbm, rel_hbm,
                    out_news, out_cand, out_small,
                    uidx_v, rs_v, cidx_v, nbuf, rbuf, sem_g):
    wid = lax.axis_index("s") * NC + lax.axis_index("c")
    ubase = pl.multiple_of(wid * UPT, 8)
    cbase = pl.multiple_of(wid * CPT, 8)

    pltpu.sync_copy(uidx_hbm.at[pl.ds(ubase, UPT)], uidx_v)
    pltpu.async_copy(ripple_hbm.at[uidx_v], rs_v, sem_g).wait()
    pltpu.sync_copy(cand_hbm.at[pl.ds(cbase, CPT)], cidx_v)

    def user_body(u, carry):
        cps = [pltpu.async_copy(news_hbm.at[rs_v.at[u, 0, 0]],
                                nbuf, sem_g)]
        for p, (tab, hop, role) in enumerate(_ROLES):
            src = ent_hbm if tab == 'ent' else rel_hbm
            cps.append(pltpu.async_copy(src.at[rs_v.at[u, hop, role]],
                                        rbuf.at[p], sem_g))
        for c in cps:
            c.wait()
        row = ubase + u
        pltpu.sync_copy(nbuf, out_news.at[pl.ds(row * NMEM, NMEM)])
        pltpu.sync_copy(rbuf, out_small.at[row])
        return carry

    lax.fori_loop(0, UPT, user_body, 0)

    def cand_body(c, carry):
        off = pl.multiple_of(c * NMEM, 8)
        pltpu.async_copy(news_hbm.at[cidx_v.at[pl.ds(off, NMEM)]],
                         nbuf, sem_g).wait()
        pltpu.sync_copy(nbuf, out_cand.at[pl.ds(cbase + c * NMEM, NMEM)])
        return carry

    lax.fori_loop(0, NCAND, cand_body, 0)


@functools.cache
def _get_sc_gather():
    return functools.partial(
        pl.kernel,
        out_type=(
            jax.ShapeDtypeStruct((B * NMEM, TITLE), jnp.float32),
            jax.ShapeDtypeStruct((B * NCAND, TITLE), jnp.float32),
            jax.ShapeDtypeStruct((B, 8, NMEM, DIM), jnp.float32),
        ),
        mesh=plsc.VectorSubcoreMesh(core_axis_name="c", subcore_axis_name="s",
                                    num_cores=NC, num_subcores=NS),
        scratch_types=[
            pltpu.VMEM((UPT,), jnp.int32),
            pltpu.VMEM((UPT, NHOP + 1, 3, NMEM), jnp.int32),
            pltpu.VMEM((CPT,), jnp.int32),
            pltpu.VMEM((NMEM, TITLE), jnp.float32),
            pltpu.VMEM((8, NMEM, DIM), jnp.float32),
            pltpu.SemaphoreType.DMA,
        ],
        compiler_params=pltpu.CompilerParams(use_tc_tiling_on_sc=False),
    )(_sc_gather_body)


BB = 64
GRID = B // BB


def _tc_body(h0rows_ref, cand_ref, small_ref, labels_ref, wn_ref, bn_ref,
             wt_ref, scores_ref, acc_ref):
    i = pl.program_id(0)
    wn = wn_ref[...]
    bn = bn_ref[...]
    wt = wt_ref[...]

    h0 = jnp.tanh(
        jnp.dot(h0rows_ref[...], wn, preferred_element_type=jnp.float32) + bn
    ).reshape(BB, NMEM, DIM)
    ne = jnp.tanh(
        jnp.dot(cand_ref[...], wn, preferred_element_type=jnp.float32) + bn
    ).reshape(BB, NCAND, DIM)

    small = small_ref[...]
    hs = [h0, small[:, 0], small[:, 1]]
    rs = [small[:, 2], small[:, 3], small[:, 4]]
    ts = [small[:, 5], small[:, 6], small[:, 7]]

    o_sum = jnp.zeros((BB, NCAND, DIM), jnp.float32)
    for hop in range(NHOP + 1):
        Rh = rs[hop] * hs[hop]
        probs = jnp.stack(
            [jnp.sum(Rh * ne[:, c][:, None, :], axis=-1) for c in range(NCAND)],
            axis=1)
        m = jnp.max(probs, axis=1, keepdims=True)
        e = jnp.exp(probs - m)
        pn = e / jnp.sum(e, axis=1, keepdims=True)
        o = jnp.stack(
            [jnp.sum(ts[hop] * pn[:, c][:, :, None], axis=1)
             for c in range(NCAND)], axis=1)
        ne = jnp.dot((ne + o).reshape(BB * NCAND, DIM), wt,
                     preferred_element_type=jnp.float32).reshape(BB, NCAND, DIM)
        o_sum = o_sum + o

    scores = jnp.sum(ne * o_sum, axis=-1)
    scores_ref[...] = scores

    lab = labels_ref[...]
    cidx = lax.broadcasted_iota(jnp.int32, (BB, NCAND), 1)
    lmax = jnp.max(lab, axis=1, keepdims=True)
    tgt = jnp.min(jnp.where(lab >= lmax, cidx, NCAND), axis=1, keepdims=True)
    sc_t = jnp.sum(jnp.where(cidx == tgt, scores, 0.0), axis=1)
    smax = jnp.max(scores, axis=1)
    lse = smax + jnp.log(jnp.sum(jnp.exp(scores - smax[:, None]), axis=1))
    nll = jnp.sum(lse - sc_t)

    def sig_sum(x):
        return jnp.sum(1.0 / (1.0 + jnp.exp(-x)))

    kge0 = sig_sum(hs[0] * rs[0] * ts[0])
    kge1 = sig_sum(hs[1] * rs[1] * ts[1])
    l2 = (jnp.sum(hs[0] * hs[0]) + jnp.sum(hs[1] * hs[1])
          + jnp.sum(rs[0] * rs[0]) + jnp.sum(rs[1] * rs[1])
          + jnp.sum(ts[0] * ts[0]) + jnp.sum(ts[1] * ts[1]))

    row = lax.broadcasted_iota(jnp.int32, (8, 128), 0)
    lane = lax.broadcasted_iota(jnp.int32, (8, 128), 1)
    contrib = jnp.zeros((8, 128), jnp.float32)
    for k, v in enumerate([nll, kge0, kge1, l2]):
        contrib = contrib + jnp.where((row == k) & (lane == 0), v, 0.0)

    @pl.when(i == 0)
    def _init():
        acc_ref[...] = jnp.zeros((8, 128), jnp.float32)

    acc_ref[...] += contrib


_tc_compute = pl.pallas_call(
    _tc_body,
    grid=(GRID,),
    in_specs=[
        pl.BlockSpec((BB * NMEM, TITLE), lambda i: (i, 0)),
        pl.BlockSpec((BB * NCAND, TITLE), lambda i: (i, 0)),
        pl.BlockSpec((BB, 8, NMEM, DIM), lambda i: (i, 0, 0, 0)),
        pl.BlockSpec((BB, NCAND), lambda i: (i, 0)),
        pl.BlockSpec((TITLE, DIM), lambda i: (0, 0)),
        pl.BlockSpec((1, DIM), lambda i: (0, 0)),
        pl.BlockSpec((DIM, DIM), lambda i: (0, 0)),
    ],
    out_specs=[
        pl.BlockSpec((BB, NCAND), lambda i: (i, 0)),
        pl.BlockSpec((8, 128), lambda i: (0, 0)),
    ],
    out_shape=[
        jax.ShapeDtypeStruct((B, NCAND), jnp.float32),
        jax.ShapeDtypeStruct((8, 128), jnp.float32),
    ],
)


def kernel(user_index, candidate_newsindex, labels, ripple_set, news_table,
           entity_table, relation_table, W_transform, W_n2e, b_n2e):
    cand_flat = candidate_newsindex.reshape(B * NCAND).astype(jnp.int32)
    out_news, out_cand, out_small = _get_sc_gather()(
        user_index.astype(jnp.int32), cand_flat, ripple_set.astype(jnp.int32),
        news_table, entity_table, relation_table)

    scores, acc = _tc_compute(
        out_news, out_cand, out_small, labels,
        W_n2e, b_n2e.reshape(1, DIM), W_transform)

    col = acc[:, 0]
    denom = float(B * NMEM * DIM)
    base_loss = col[0] / B
    kge_loss = -KGE_W * (col[1] / denom + col[2] / denom)
    l2_loss = L2_W * col[3]
    loss = base_loss + kge_loss + l2_loss
    return (base_loss, kge_loss, l2_loss, loss, scores)


# speedup vs baseline: 1.8726x; 1.8726x over previous
import functools

import jax
import jax.numpy as jnp
from jax import lax
from jax.experimental import pallas as pl
from jax.experimental.pallas import tpu as pltpu
from jax.experimental.pallas import tpu_sc as plsc

B = 1024
NUSERS = 16384
NNEWS = 100001
DIM = 64
TITLE = 768
NHOP = 2
NMEM = 32
KGE_W = 0.01
L2_W = 1e-07

NC = 2
NS = 16
NW = NC * NS
UPT = B // NW
NCAND = 5
CPT = NCAND * UPT
RPT = UPT * NMEM

_ROLES = (
    ('ent', (1 * 3 + 0) * NMEM),
    ('ent', (2 * 3 + 0) * NMEM),
    ('rel', (0 * 3 + 1) * NMEM),
    ('rel', (1 * 3 + 1) * NMEM),
    ('rel', (2 * 3 + 1) * NMEM),
    ('ent', (0 * 3 + 2) * NMEM),
    ('ent', (1 * 3 + 2) * NMEM),
    ('ent', (2 * 3 + 2) * NMEM),
)

ECH = 128
NECH = RPT // ECH
NCH = 64
NNCH = RPT // NCH


def _k1_body(uidx_hbm, ripple_hbm, ent_hbm, rel_hbm,
             out_small, out_nidx,
             uidx_v, rs_v, ridx_v, nidx_v, ebuf, sem_g, sem_w):
    wid = lax.axis_index("s") * NC + lax.axis_index("c")
    ubase = pl.multiple_of(wid * UPT, 8)

    pltpu.sync_copy(uidx_hbm.at[pl.ds(ubase, UPT)], uidx_v)
    pltpu.async_copy(ripple_hbm.at[uidx_v], rs_v, sem_g).wait()

    def build(u, carry):
        for c in range(NMEM // 16):
            dst = pl.multiple_of(u * NMEM + c * 16, 16)
            v = rs_v[u, pl.ds(c * 16, 16)]
            nidx_v[pl.ds(dst, 16)] = v
            for p, (_, off) in enumerate(_ROLES):
                ridx_v[p, pl.ds(dst, 16)] = rs_v[u, pl.ds(off + c * 16, 16)]
        return carry

    lax.fori_loop(0, UPT, build, 0)

    pltpu.sync_copy(nidx_v, out_nidx.at[pl.ds(wid * RPT, RPT)])

    seq = [(p, ch) for p in range(8) for ch in range(NECH)]
    gd = {}
    wd = {}
    for j, (p, ch) in enumerate(seq):
        b = j % 2
        if j >= 2:
            wd[j - 2].wait()
        src = ent_hbm if _ROLES[p][0] == 'ent' else rel_hbm
        gd[j] = pltpu.async_copy(
            src.at[ridx_v.at[p, pl.ds(ch * ECH, ECH)]], ebuf.at[b], sem_g)
        if j >= 1:
            pj, pch = seq[j - 1]
            gd[j - 1].wait()
            wd[j - 1] = pltpu.async_copy(
                ebuf.at[(j - 1) % 2],
                out_small.at[pj, pl.ds(wid * RPT + pch * ECH, ECH)], sem_w)
    j = len(seq) - 1
    gd[j].wait()
    wd[j] = pltpu.async_copy(
        ebuf.at[j % 2],
        out_small.at[seq[j][0], pl.ds(wid * RPT + seq[j][1] * ECH, ECH)],
        sem_w)
    wd[j - 1].wait()
    wd[j].wait()


def _k2_body(nidx_hbm, cand_hbm, news_hbm, out_news, out_cand,
             nidx_v, cidx_v, nbuf, sem_g, sem_w):
    wid = lax.axis_index("s") * NC + lax.axis_index("c")
    nbase = pl.multiple_of(wid * RPT, 8)
    cbase = pl.multiple_of(wid * CPT, 8)

    pltpu.sync_copy(nidx_hbm.at[pl.ds(nbase, RPT)], nidx_v)
    pltpu.sync_copy(cand_hbm.at[pl.ds(cbase, CPT)], cidx_v)

    seq = [('n', ch) for ch in range(NNCH)] + [('c', ch) for ch in range(NCAND)]

    def fire(j):
        kind, ch = seq[j]
        b = j % 2
        if kind == 'n':
            return pltpu.async_copy(
                news_hbm.at[nidx_v.at[pl.ds(ch * NCH, NCH)]],
                nbuf.at[b], sem_g)
        return pltpu.async_copy(
            news_hbm.at[cidx_v.at[pl.ds(ch * NMEM, NMEM)]],
            nbuf.at[b, pl.ds(0, NMEM)], sem_g)

    def drain(j):
        kind, ch = seq[j]
        b = j % 2
        if kind == 'n':
            return pltpu.async_copy(
                nbuf.at[b], out_news.at[pl.ds(nbase + ch * NCH, NCH)], sem_w)
        return pltpu.async_copy(
            nbuf.at[b, pl.ds(0, NMEM)],
            out_cand.at[pl.ds(cbase + ch * NMEM, NMEM)], sem_w)

    gd = {}
    wd = {}
    for j in range(len(seq)):
        if j >= 2:
            wd[j - 2].wait()
        gd[j] = fire(j)
        if j >= 1:
            gd[j - 1].wait()
            wd[j - 1] = drain(j - 1)
    j = len(seq) - 1
    gd[j].wait()
    wd[j] = drain(j)
    wd[j - 1].wait()
    wd[j].wait()


@functools.cache
def _get_k1():
    return functools.partial(
        pl.kernel,
        out_type=(
            jax.ShapeDtypeStruct((8, B * NMEM, DIM), jnp.float32),
            jax.ShapeDtypeStruct((B * NMEM,), jnp.int32),
        ),
        mesh=plsc.VectorSubcoreMesh(core_axis_name="c", subcore_axis_name="s",
                                    num_cores=NC, num_subcores=NS),
        scratch_types=[
            pltpu.VMEM((UPT,), jnp.int32),
            pltpu.VMEM((UPT, 9 * NMEM), jnp.int32),
            pltpu.VMEM((8, RPT), jnp.int32),
            pltpu.VMEM((RPT,), jnp.int32),
            pltpu.VMEM((2, ECH, DIM), jnp.float32),
            pltpu.SemaphoreType.DMA,
            pltpu.SemaphoreType.DMA,
        ],
        compiler_params=pltpu.CompilerParams(use_tc_tiling_on_sc=False),
    )(_k1_body)


@functools.cache
def _get_k2():
    return functools.partial(
        pl.kernel,
        out_type=(
            jax.ShapeDtypeStruct((B * NMEM, TITLE), jnp.float32),
            jax.ShapeDtypeStruct((B * NCAND, TITLE), jnp.float32),
        ),
        mesh=plsc.VectorSubcoreMesh(core_axis_name="c", subcore_axis_name="s",
                                    num_cores=NC, num_subcores=NS),
        scratch_types=[
            pltpu.VMEM((RPT,), jnp.int32),
            pltpu.VMEM((CPT,), jnp.int32),
            pltpu.VMEM((2, NCH, TITLE), jnp.float32),
            pltpu.SemaphoreType.DMA,
            pltpu.SemaphoreType.DMA,
        ],
        compiler_params=pltpu.CompilerParams(use_tc_tiling_on_sc=True),
    )(_k2_body)


BB = 64
GRID = B // BB


def _tc_body(h0rows_ref, cand_ref, small_ref, labels_ref, wn_ref, bn_ref,
             wt_ref, scores_ref, acc_ref):
    i = pl.program_id(0)
    wn = wn_ref[...]
    bn = bn_ref[...]
    wt = wt_ref[...]

    h0 = jnp.tanh(
        jnp.dot(h0rows_ref[...], wn, preferred_element_type=jnp.float32) + bn
    ).reshape(BB, NMEM, DIM)
    ne = jnp.tanh(
        jnp.dot(cand_ref[...], wn, preferred_element_type=jnp.float32) + bn
    ).reshape(BB, NCAND, DIM)

    small = small_ref[...]
    hs = [h0, small[0].reshape(BB, NMEM, DIM), small[1].reshape(BB, NMEM, DIM)]
    rs = [small[2].reshape(BB, NMEM, DIM), small[3].reshape(BB, NMEM, DIM),
          small[4].reshape(BB, NMEM, DIM)]
    ts = [small[5].reshape(BB, NMEM, DIM), small[6].reshape(BB, NMEM, DIM),
          small[7].reshape(BB, NMEM, DIM)]

    o_sum = jnp.zeros((BB, NCAND, DIM), jnp.float32)
    for hop in range(NHOP + 1):
        Rh = rs[hop] * hs[hop]
        probs = jnp.stack(
            [jnp.sum(Rh * ne[:, c][:, None, :], axis=-1) for c in range(NCAND)],
            axis=1)
        m = jnp.max(probs, axis=1, keepdims=True)
        e = jnp.exp(probs - m)
        pn = e / jnp.sum(e, axis=1, keepdims=True)
        o = jnp.stack(
            [jnp.sum(ts[hop] * pn[:, c][:, :, None], axis=1)
             for c in range(NCAND)], axis=1)
        ne = jnp.dot((ne + o).reshape(BB * NCAND, DIM), wt,
                     preferred_element_type=jnp.float32).reshape(BB, NCAND, DIM)
        o_sum = o_sum + o

    scores = jnp.sum(ne * o_sum, axis=-1)
    scores_ref[...] = scores

    lab = labels_ref[...]
    cidx = lax.broadcasted_iota(jnp.int32, (BB, NCAND), 1)
    lmax = jnp.max(lab, axis=1, keepdims=True)
    tgt = jnp.min(jnp.where(lab >= lmax, cidx, NCAND), axis=1, keepdims=True)
    sc_t = jnp.sum(jnp.where(cidx == tgt, scores, 0.0), axis=1)
    smax = jnp.max(scores, axis=1)
    lse = smax + jnp.log(jnp.sum(jnp.exp(scores - smax[:, None]), axis=1))
    nll = jnp.sum(lse - sc_t)

    def sig_sum(x):
        return jnp.sum(1.0 / (1.0 + jnp.exp(-x)))

    kge0 = sig_sum(hs[0] * rs[0] * ts[0])
    kge1 = sig_sum(hs[1] * rs[1] * ts[1])
    l2 = (jnp.sum(hs[0] * hs[0]) + jnp.sum(hs[1] * hs[1])
          + jnp.sum(rs[0] * rs[0]) + jnp.sum(rs[1] * rs[1])
          + jnp.sum(ts[0] * ts[0]) + jnp.sum(ts[1] * ts[1]))

    row = lax.broadcasted_iota(jnp.int32, (8, 128), 0)
    lane = lax.broadcasted_iota(jnp.int32, (8, 128), 1)
    contrib = jnp.zeros((8, 128), jnp.float32)
    for k, v in enumerate([nll, kge0, kge1, l2]):
        contrib = contrib + jnp.where((row == k) & (lane == 0), v, 0.0)

    @pl.when(i == 0)
    def _init():
        acc_ref[...] = jnp.zeros((8, 128), jnp.float32)

    acc_ref[...] += contrib


_tc_compute = pl.pallas_call(
    _tc_body,
    grid=(GRID,),
    in_specs=[
        pl.BlockSpec((BB * NMEM, TITLE), lambda i: (i, 0)),
        pl.BlockSpec((BB * NCAND, TITLE), lambda i: (i, 0)),
        pl.BlockSpec((8, BB * NMEM, DIM), lambda i: (0, i, 0)),
        pl.BlockSpec((BB, NCAND), lambda i: (i, 0)),
        pl.BlockSpec((TITLE, DIM), lambda i: (0, 0)),
        pl.BlockSpec((1, DIM), lambda i: (0, 0)),
        pl.BlockSpec((DIM, DIM), lambda i: (0, 0)),
    ],
    out_specs=[
        pl.BlockSpec((BB, NCAND), lambda i: (i, 0)),
        pl.BlockSpec((8, 128), lambda i: (0, 0)),
    ],
    out_shape=[
        jax.ShapeDtypeStruct((B, NCAND), jnp.float32),
        jax.ShapeDtypeStruct((8, 128), jnp.float32),
    ],
)


def kernel(user_index, candidate_newsindex, labels, ripple_set, news_table,
           entity_table, relation_table, W_transform, W_n2e, b_n2e):
    ripple2d = ripple_set.reshape(NUSERS, 9 * NMEM).astype(jnp.int32)
    cand_flat = candidate_newsindex.reshape(B * NCAND).astype(jnp.int32)

    out_small, out_nidx = _get_k1()(
        user_index.astype(jnp.int32), ripple2d, entity_table, relation_table)
    out_news, out_cand = _get_k2()(out_nidx, cand_flat, news_table)

    scores, acc = _tc_compute(
        out_news, out_cand, out_small, labels,
        W_n2e, b_n2e.reshape(1, DIM), W_transform)

    col = acc[:, 0]
    denom = float(B * NMEM * DIM)
    base_loss = col[0] / B
    kge_loss = -KGE_W * (col[1] / denom + col[2] / denom)
    l2_loss = L2_W * col[3]
    loss = base_loss + kge_loss + l2_loss
    return (base_loss, kge_loss, l2_loss, loss, scores)
